# Initial kernel scaffold; baseline (speedup 1.0000x reference)
import jax
import jax.numpy as jnp
from jax.experimental import pallas as pl


def kernel(x, edge_index, edge_attr, batch, params):
    raise NotImplementedError("write your pallas kernel here")


# speedup vs baseline: 4.0864x; 4.0864x over previous
import functools

import jax
import jax.numpy as jnp
from jax import lax
from jax.experimental import pallas as pl
from jax.experimental.pallas import tpu as pltpu
from jax.experimental.pallas import tpu_sc as plsc

N = 10000
E = 640000
G = 64
FEA = 16
N_LAYERS = 4
NEG_SLOPE = 0.01

NW = 32
CHUNK = 80
SCH = 25
EPW = E // NW
NSCH = EPW // (SCH * CHUNK)
ROWS_OUT = 632


def _lrelu(v):
    return jnp.where(v >= 0, v, NEG_SLOPE * v)


def _node_prep_body(x_ref, nwT_ref, nb_ref, wd_ref, ws_ref,
                    h_ref, pdqd_ref, psqs_ref):
    h = _lrelu(jnp.dot(x_ref[...], nwT_ref[...],
                       preferred_element_type=jnp.float32) + nb_ref[...])
    h_ref[...] = h
    pdqd_ref[...] = jnp.dot(h, wd_ref[...], preferred_element_type=jnp.float32)
    psqs_ref[...] = jnp.dot(h, ws_ref[...], preferred_element_type=jnp.float32)


def _edge_prep_body(eattr_ref, ewT_ref, eb_ref, wfeT_ref, bf_ref, wseT_ref,
                    bs_ref, f_ref, s_ref):
    ea = _lrelu(jnp.dot(eattr_ref[...], ewT_ref[...],
                        preferred_element_type=jnp.float32) + eb_ref[...])
    f_ref[...] = jnp.dot(ea, wfeT_ref[...],
                         preferred_element_type=jnp.float32) + bf_ref[...]
    s_ref[...] = jnp.dot(ea, wseT_ref[...],
                         preferred_element_type=jnp.float32) + bs_ref[...]


def _norm_res(aggp_ref, cntp_ref, h_ref, gamma_ref, beta_ref):
    cnt = jnp.maximum(cntp_ref[0] + cntp_ref[1], 1.0)
    agg = (aggp_ref[0] + aggp_ref[1]) / cnt
    mu = jnp.mean(agg, axis=0, keepdims=True)
    var = jnp.mean((agg - mu) ** 2, axis=0, keepdims=True)
    normed = gamma_ref[...] * (agg - mu) / jnp.sqrt(var + 1e-5) + beta_ref[...]
    return normed + h_ref[...]


def _post_body(aggp_ref, cntp_ref, h_ref, gamma_ref, beta_ref, wd_ref, ws_ref,
               hout_ref, pdqd_ref, psqs_ref):
    h2 = _norm_res(aggp_ref, cntp_ref, h_ref, gamma_ref, beta_ref)
    hout_ref[...] = h2
    pdqd_ref[...] = jnp.dot(h2, wd_ref[...], preferred_element_type=jnp.float32)
    psqs_ref[...] = jnp.dot(h2, ws_ref[...], preferred_element_type=jnp.float32)


def _post_pool_body(aggp_ref, cntp_ref, h_ref, gamma_ref, beta_ref, batch_ref,
                    gw1T_ref, gb1_ref, gw2T_ref, gb2_ref,
                    nw1T_ref, nb1_ref, nw2T_ref, nb2_ref,
                    hwT_ref, hb_ref, owT_ref, ob_ref, y_ref):
    h2 = _norm_res(aggp_ref, cntp_ref, h_ref, gamma_ref, beta_ref)
    gate = jnp.dot(jnp.dot(h2, gw1T_ref[...],
                           preferred_element_type=jnp.float32) + gb1_ref[...],
                   gw2T_ref[...], preferred_element_type=jnp.float32) + gb2_ref[...]
    v = jnp.dot(jnp.dot(h2, nw1T_ref[...],
                        preferred_element_type=jnp.float32) + nb1_ref[...],
                nw2T_ref[...], preferred_element_type=jnp.float32) + nb2_ref[...]
    oh = batch_ref[...] == lax.broadcasted_iota(jnp.int32, (N, G), 1)
    ohf = oh.astype(jnp.float32)
    gmax = jnp.max(jnp.where(oh, gate, -1e30), axis=0, keepdims=True)
    gmax_n = jnp.dot(ohf, jnp.reshape(gmax, (G, 1)),
                     preferred_element_type=jnp.float32)
    eg = jnp.exp(gate - gmax_n)
    den = lax.dot_general(ohf, eg, (((0,), (0,)), ((), ())),
                          preferred_element_type=jnp.float32)
    den_n = jnp.dot(ohf, den, preferred_element_type=jnp.float32)
    pooled = lax.dot_general(ohf, (eg / den_n) * v, (((0,), (0,)), ((), ())),
                             preferred_element_type=jnp.float32)
    hh = _lrelu(jnp.dot(pooled, hwT_ref[...],
                        preferred_element_type=jnp.float32) + hb_ref[...])
    y_ref[...] = jnp.dot(hh, owT_ref[...],
                         preferred_element_type=jnp.float32) + ob_ref[...]


_SC_MESH = plsc.VectorSubcoreMesh(core_axis_name="c", subcore_axis_name="s")


def _acc_off(sid):
    return pl.multiple_of(jnp.minimum(sid * ROWS_OUT, N - ROWS_OUT), 8)


def _zero_acc(sid, zBuf, agg):
    def _zrow(k, _):
        zBuf[k] = jnp.zeros((FEA,), jnp.float32)
        return 0
    lax.fori_loop(0, ROWS_OUT, _zrow, 0)
    pltpu.sync_copy(zBuf, agg.at[pl.ds(_acc_off(sid), ROWS_OUT)])


def _read_acc(cid, sid, zBuf, agg, out):
    off = _acc_off(sid)
    pltpu.sync_copy(agg.at[pl.ds(off, ROWS_OUT)], zBuf)
    pltpu.sync_copy(zBuf, out.at[cid, pl.ds(off, ROWS_OUT)])


@functools.partial(
    pl.kernel,
    mesh=_SC_MESH,
    out_type=jax.ShapeDtypeStruct((2, N, FEA), jnp.float32),
    compiler_params=pltpu.CompilerParams(use_tc_tiling_on_sc=False),
    scratch_types=[
        pltpu.VMEM((SCH, CHUNK), jnp.int32),
        pltpu.VMEM((SCH, CHUNK), jnp.int32),
        pltpu.VMEM((CHUNK, 2 * FEA), jnp.float32),
        pltpu.VMEM((CHUNK, 2 * FEA), jnp.float32),
        pltpu.VMEM((CHUNK, FEA), jnp.float32),
        pltpu.VMEM((CHUNK, FEA), jnp.float32),
        pltpu.VMEM((CHUNK, FEA), jnp.float32),
        pltpu.VMEM((ROWS_OUT, FEA), jnp.float32),
        pltpu.VMEM_SHARED((N, FEA), jnp.float32),
        pltpu.SemaphoreType.DMA,
        pltpu.SemaphoreType.DMA,
    ],
)
def _sc_layer(dst3d, src3d, pdqd, psqs, fterm, sterm, out,
              dIdx, sIdx, dBuf, sBuf, fBuf, gBuf, mBuf, zBuf,
              agg, semD, semS):
    cid = lax.axis_index("c")
    sid = lax.axis_index("s")
    wid = sid * 2 + cid

    _zero_acc(sid, zBuf, agg)
    plsc.subcore_barrier()

    t0 = wid * NSCH

    def _schunk(t, _):
        pltpu.sync_copy(dst3d.at[t0 + t], dIdx)
        pltpu.sync_copy(src3d.at[t0 + t], sIdx)

        def _chunk(k, _):
            cd = pltpu.async_copy(pdqd.at[dIdx.at[k]], dBuf, semD)
            cs = pltpu.async_copy(psqs.at[sIdx.at[k]], sBuf, semS)
            e0 = ((t0 + t) * SCH + k) * CHUNK
            pltpu.sync_copy(fterm.at[pl.ds(e0, CHUNK)], fBuf)
            pltpu.sync_copy(sterm.at[pl.ds(e0, CHUNK)], gBuf)
            cd.wait()
            cs.wait()

            def _edge(i, _):
                gf = dBuf[i, 0:FEA] + sBuf[i, 0:FEA] + fBuf[i]
                gs = dBuf[i, FEA:2 * FEA] + sBuf[i, FEA:2 * FEA] + gBuf[i]
                sig = 1.0 / (1.0 + jnp.exp(-gf))
                t_ = jnp.exp(-jnp.abs(gs))
                z = t_ / (2.0 + t_)
                z2 = z * z
                ln = (2.0 * z) * (1.0 + z2 * (
                    1.0 / 3.0 + z2 * (1.0 / 5.0 + z2 * (
                        1.0 / 7.0 + z2 * (1.0 / 9.0 + z2 * (1.0 / 11.0))))))
                sp = jnp.maximum(gs, 0.0) + ln
                mBuf[i] = sig * sp
                return 0
            lax.fori_loop(0, CHUNK, _edge, 0)
            pltpu.sync_copy(mBuf, agg.at[dIdx.at[k]], add=True)
            return 0
        lax.fori_loop(0, SCH, _chunk, 0)
        return 0
    lax.fori_loop(0, NSCH, _schunk, 0)

    plsc.subcore_barrier()
    _read_acc(cid, sid, zBuf, agg, out)


@functools.partial(
    pl.kernel,
    mesh=_SC_MESH,
    out_type=jax.ShapeDtypeStruct((2, N, FEA), jnp.float32),
    compiler_params=pltpu.CompilerParams(use_tc_tiling_on_sc=False),
    scratch_types=[
        pltpu.VMEM((SCH, CHUNK), jnp.int32),
        pltpu.VMEM((CHUNK, FEA), jnp.float32),
        pltpu.VMEM((ROWS_OUT, FEA), jnp.float32),
        pltpu.VMEM_SHARED((N, FEA), jnp.float32),
    ],
)
def _sc_cnt(dst3d, out, dIdx, oBuf, zBuf, agg):
    cid = lax.axis_index("c")
    sid = lax.axis_index("s")
    wid = sid * 2 + cid

    _zero_acc(sid, zBuf, agg)

    def _orow(k, _):
        oBuf[k] = jnp.ones((FEA,), jnp.float32)
        return 0
    lax.fori_loop(0, CHUNK, _orow, 0)
    plsc.subcore_barrier()

    t0 = wid * NSCH

    def _schunk(t, _):
        pltpu.sync_copy(dst3d.at[t0 + t], dIdx)

        def _chunk(k, _):
            pltpu.sync_copy(oBuf, agg.at[dIdx.at[k]], add=True)
            return 0
        lax.fori_loop(0, SCH, _chunk, 0)
        return 0
    lax.fori_loop(0, NSCH, _schunk, 0)

    plsc.subcore_barrier()
    _read_acc(cid, sid, zBuf, agg, out)


def _row(b):
    return jnp.reshape(b, (1, -1))


def kernel(x, edge_index, edge_attr, batch, params):
    p = params
    convs = p['convs']
    node_in = x.shape[1]
    edge_in = edge_attr.shape[1]

    src3d = edge_index[0].reshape(E // (SCH * CHUNK), SCH, CHUNK)
    dst3d = edge_index[1].reshape(E // (SCH * CHUNK), SCH, CHUNK)

    wd_l = [jnp.concatenate([c['wf'][:, :FEA].T, c['ws'][:, :FEA].T], axis=1)
            for c in convs]
    ws_l = [jnp.concatenate([c['wf'][:, FEA:2 * FEA].T,
                             c['ws'][:, FEA:2 * FEA].T], axis=1)
            for c in convs]

    BLK_N = 2000
    h0, pdqd, psqs = pl.pallas_call(
        _node_prep_body,
        grid=(N // BLK_N,),
        in_specs=[pl.BlockSpec((BLK_N, node_in), lambda i: (i, 0)),
                  pl.BlockSpec((node_in, FEA), lambda i: (0, 0)),
                  pl.BlockSpec((1, FEA), lambda i: (0, 0)),
                  pl.BlockSpec((FEA, 2 * FEA), lambda i: (0, 0)),
                  pl.BlockSpec((FEA, 2 * FEA), lambda i: (0, 0))],
        out_specs=[pl.BlockSpec((BLK_N, FEA), lambda i: (i, 0)),
                   pl.BlockSpec((BLK_N, 2 * FEA), lambda i: (i, 0)),
                   pl.BlockSpec((BLK_N, 2 * FEA), lambda i: (i, 0))],
        out_shape=[jax.ShapeDtypeStruct((N, FEA), jnp.float32),
                   jax.ShapeDtypeStruct((N, 2 * FEA), jnp.float32),
                   jax.ShapeDtypeStruct((N, 2 * FEA), jnp.float32)],
    )(x, p['node_w'].T, _row(p['node_b']), wd_l[0], ws_l[0])

    BLK_E = 4000
    fs = []
    for c in convs:
        f_l, s_l = pl.pallas_call(
            _edge_prep_body,
            grid=(E // BLK_E,),
            in_specs=[pl.BlockSpec((BLK_E, edge_in), lambda i: (i, 0)),
                      pl.BlockSpec((edge_in, FEA), lambda i: (0, 0)),
                      pl.BlockSpec((1, FEA), lambda i: (0, 0)),
                      pl.BlockSpec((FEA, FEA), lambda i: (0, 0)),
                      pl.BlockSpec((1, FEA), lambda i: (0, 0)),
                      pl.BlockSpec((FEA, FEA), lambda i: (0, 0)),
                      pl.BlockSpec((1, FEA), lambda i: (0, 0))],
            out_specs=[pl.BlockSpec((BLK_E, FEA), lambda i: (i, 0)),
                       pl.BlockSpec((BLK_E, FEA), lambda i: (i, 0))],
            out_shape=[jax.ShapeDtypeStruct((E, FEA), jnp.float32),
                       jax.ShapeDtypeStruct((E, FEA), jnp.float32)],
        )(edge_attr, p['edge_w'].T, _row(p['edge_b']),
          c['wf'][:, 2 * FEA:].T, _row(c['bf']),
          c['ws'][:, 2 * FEA:].T, _row(c['bs']))
        fs.append((f_l, s_l))

    cntp = _sc_cnt(dst3d)

    h = h0
    for l in range(N_LAYERS):
        c = convs[l]
        aggp = _sc_layer(dst3d, src3d, pdqd, psqs, fs[l][0], fs[l][1])
        if l < N_LAYERS - 1:
            h, pdqd, psqs = pl.pallas_call(
                _post_body,
                out_shape=[jax.ShapeDtypeStruct((N, FEA), jnp.float32),
                           jax.ShapeDtypeStruct((N, 2 * FEA), jnp.float32),
                           jax.ShapeDtypeStruct((N, 2 * FEA), jnp.float32)],
            )(aggp, cntp, h, _row(c['gamma']), _row(c['beta']),
              wd_l[l + 1], ws_l[l + 1])
        else:
            y = pl.pallas_call(
                _post_pool_body,
                out_shape=jax.ShapeDtypeStruct((G, 1), jnp.float32),
            )(aggp, cntp, h, _row(c['gamma']), _row(c['beta']),
              batch.reshape(N, 1),
              p['gate_w1'].T, _row(p['gate_b1']),
              p['gate_w2'].T, _row(p['gate_b2']),
              p['nn_w1'].T, _row(p['nn_b1']),
              p['nn_w2'].T, _row(p['nn_b2']),
              p['h_w'].T, _row(p['h_b']),
              p['out_w'].T, _row(p['out_b']))
    return jnp.reshape(y, (G,))


# speedup vs baseline: 6.6636x; 1.6307x over previous
import functools

import jax
import jax.numpy as jnp
from jax import lax
from jax.experimental import pallas as pl
from jax.experimental.pallas import tpu as pltpu
from jax.experimental.pallas import tpu_sc as plsc

N = 10000
E = 640000
G = 64
FEA = 16
N_LAYERS = 4
NEG_SLOPE = 0.01

NW = 32
CHUNK = 80
SCH = 25
EPW = E // NW
NSCH = EPW // (SCH * CHUNK)
ROWS_OUT = 632


def _lrelu(v):
    return jnp.where(v >= 0, v, NEG_SLOPE * v)


def _node_prep_body(x_ref, nwT_ref, nb_ref, wd_ref, ws_ref,
                    h_ref, pdqd_ref, psqs_ref):
    h = _lrelu(jnp.dot(x_ref[...], nwT_ref[...],
                       preferred_element_type=jnp.float32) + nb_ref[...])
    h_ref[...] = h
    pdqd_ref[...] = jnp.dot(h, wd_ref[...], preferred_element_type=jnp.float32)
    psqs_ref[...] = jnp.dot(h, ws_ref[...], preferred_element_type=jnp.float32)


def _edge_prep_body(eattr_ref, ewT_ref, eb_ref, wfeT_ref, bf_ref, wseT_ref,
                    bs_ref, f_ref, s_ref):
    ea = _lrelu(jnp.dot(eattr_ref[...], ewT_ref[...],
                        preferred_element_type=jnp.float32) + eb_ref[...])
    f_ref[...] = jnp.dot(ea, wfeT_ref[...],
                         preferred_element_type=jnp.float32) + bf_ref[...]
    s_ref[...] = jnp.dot(ea, wseT_ref[...],
                         preferred_element_type=jnp.float32) + bs_ref[...]


def _norm_res(aggp_ref, cntp_ref, h_ref, gamma_ref, beta_ref):
    cnt = jnp.maximum(cntp_ref[0] + cntp_ref[1], 1.0)
    agg = (aggp_ref[0] + aggp_ref[1]) / cnt
    mu = jnp.mean(agg, axis=0, keepdims=True)
    var = jnp.mean((agg - mu) ** 2, axis=0, keepdims=True)
    normed = gamma_ref[...] * (agg - mu) / jnp.sqrt(var + 1e-5) + beta_ref[...]
    return normed + h_ref[...]


def _post_body(aggp_ref, cntp_ref, h_ref, gamma_ref, beta_ref, wd_ref, ws_ref,
               hout_ref, pdqd_ref, psqs_ref):
    h2 = _norm_res(aggp_ref, cntp_ref, h_ref, gamma_ref, beta_ref)
    hout_ref[...] = h2
    pdqd_ref[...] = jnp.dot(h2, wd_ref[...], preferred_element_type=jnp.float32)
    psqs_ref[...] = jnp.dot(h2, ws_ref[...], preferred_element_type=jnp.float32)


def _post_pool_body(aggp_ref, cntp_ref, h_ref, gamma_ref, beta_ref, batch_ref,
                    gw1T_ref, gb1_ref, gw2T_ref, gb2_ref,
                    nw1T_ref, nb1_ref, nw2T_ref, nb2_ref,
                    hwT_ref, hb_ref, owT_ref, ob_ref, y_ref):
    h2 = _norm_res(aggp_ref, cntp_ref, h_ref, gamma_ref, beta_ref)
    gate = jnp.dot(jnp.dot(h2, gw1T_ref[...],
                           preferred_element_type=jnp.float32) + gb1_ref[...],
                   gw2T_ref[...], preferred_element_type=jnp.float32) + gb2_ref[...]
    v = jnp.dot(jnp.dot(h2, nw1T_ref[...],
                        preferred_element_type=jnp.float32) + nb1_ref[...],
                nw2T_ref[...], preferred_element_type=jnp.float32) + nb2_ref[...]
    oh = batch_ref[...] == lax.broadcasted_iota(jnp.int32, (N, G), 1)
    ohf = oh.astype(jnp.float32)
    gmax = jnp.max(jnp.where(oh, gate, -1e30), axis=0, keepdims=True)
    gmax_n = jnp.dot(ohf, jnp.reshape(gmax, (G, 1)),
                     preferred_element_type=jnp.float32)
    eg = jnp.exp(gate - gmax_n)
    den = lax.dot_general(ohf, eg, (((0,), (0,)), ((), ())),
                          preferred_element_type=jnp.float32)
    den_n = jnp.dot(ohf, den, preferred_element_type=jnp.float32)
    pooled = lax.dot_general(ohf, (eg / den_n) * v, (((0,), (0,)), ((), ())),
                             preferred_element_type=jnp.float32)
    hh = _lrelu(jnp.dot(pooled, hwT_ref[...],
                        preferred_element_type=jnp.float32) + hb_ref[...])
    y_ref[...] = jnp.dot(hh, owT_ref[...],
                         preferred_element_type=jnp.float32) + ob_ref[...]


_SC_MESH = plsc.VectorSubcoreMesh(core_axis_name="c", subcore_axis_name="s")


def _acc_off(sid):
    return pl.multiple_of(jnp.minimum(sid * ROWS_OUT, N - ROWS_OUT), 8)


def _zero_acc(sid, zBuf, agg):
    def _zrow(k, _):
        zBuf[k] = jnp.zeros((FEA,), jnp.float32)
        return 0
    lax.fori_loop(0, ROWS_OUT, _zrow, 0)
    pltpu.sync_copy(zBuf, agg.at[pl.ds(_acc_off(sid), ROWS_OUT)])


def _read_acc(cid, sid, zBuf, agg, out):
    off = _acc_off(sid)
    pltpu.sync_copy(agg.at[pl.ds(off, ROWS_OUT)], zBuf)
    pltpu.sync_copy(zBuf, out.at[cid, pl.ds(off, ROWS_OUT)])


DEPTH = 3
UNROLL = 4

_SC_LAYER_SCRATCH = (
    [pltpu.VMEM((SCH, CHUNK), jnp.int32)] * 2 +
    [pltpu.VMEM((CHUNK, 2 * FEA), jnp.float32)] * (2 * DEPTH) +
    [pltpu.VMEM((CHUNK, FEA), jnp.float32)] * (3 * DEPTH) +
    [pltpu.VMEM((ROWS_OUT, FEA), jnp.float32)] +
    [pltpu.VMEM_SHARED((N, FEA), jnp.float32)] +
    [pltpu.SemaphoreType.DMA] * (5 * DEPTH)
)


@functools.partial(
    pl.kernel,
    mesh=_SC_MESH,
    out_type=jax.ShapeDtypeStruct((2, N, FEA), jnp.float32),
    compiler_params=pltpu.CompilerParams(use_tc_tiling_on_sc=False),
    scratch_types=_SC_LAYER_SCRATCH,
)
def _sc_layer(dst3d, src3d, pdqd, psqs, fterm, sterm, out, *scr):
    dIdx, sIdx = scr[0], scr[1]
    dB = scr[2:2 + DEPTH]
    sB = scr[2 + DEPTH:2 + 2 * DEPTH]
    fB = scr[2 + 2 * DEPTH:2 + 3 * DEPTH]
    gB = scr[2 + 3 * DEPTH:2 + 4 * DEPTH]
    mB = scr[2 + 4 * DEPTH:2 + 5 * DEPTH]
    zBuf = scr[2 + 5 * DEPTH]
    agg = scr[3 + 5 * DEPTH]
    sems = scr[4 + 5 * DEPTH:]
    semD = sems[0:DEPTH]
    semS = sems[DEPTH:2 * DEPTH]
    semF = sems[2 * DEPTH:3 * DEPTH]
    semG = sems[3 * DEPTH:4 * DEPTH]
    semM = sems[4 * DEPTH:5 * DEPTH]

    cid = lax.axis_index("c")
    sid = lax.axis_index("s")
    wid = sid * 2 + cid

    _zero_acc(sid, zBuf, agg)
    plsc.subcore_barrier()

    t0 = wid * NSCH

    def _schunk(t, _):
        r0 = (t0 + t) * SCH
        pltpu.sync_copy(dst3d.at[t0 + t], dIdx)
        pltpu.sync_copy(src3d.at[t0 + t], sIdx)
        hD = [None] * SCH
        hS = [None] * SCH
        hF = [None] * SCH
        hG = [None] * SCH
        hM = [None] * SCH

        def _issue(k):
            b = k % DEPTH
            hD[k] = pltpu.async_copy(pdqd.at[dIdx.at[k]], dB[b], semD[b])
            hS[k] = pltpu.async_copy(psqs.at[sIdx.at[k]], sB[b], semS[b])
            e0 = (r0 + k) * CHUNK
            hF[k] = pltpu.async_copy(fterm.at[pl.ds(e0, CHUNK)], fB[b], semF[b])
            hG[k] = pltpu.async_copy(sterm.at[pl.ds(e0, CHUNK)], gB[b], semG[b])

        for k in range(DEPTH - 1):
            _issue(k)
        for k in range(SCH):
            b = k % DEPTH
            if k + DEPTH - 1 < SCH:
                _issue(k + DEPTH - 1)
            hD[k].wait()
            hS[k].wait()
            hF[k].wait()
            hG[k].wait()
            if k - DEPTH >= 0:
                hM[k - DEPTH].wait()
            dBuf, sBuf, fBuf, gBuf, mBuf = dB[b], sB[b], fB[b], gB[b], mB[b]

            @functools.partial(plsc.parallel_loop, 0, CHUNK, unroll=UNROLL)
            def _edge(i):
                gf = dBuf[i, 0:FEA] + sBuf[i, 0:FEA] + fBuf[i]
                gs = dBuf[i, FEA:2 * FEA] + sBuf[i, FEA:2 * FEA] + gBuf[i]
                sig = 1.0 / (1.0 + jnp.exp(-gf))
                t_ = jnp.exp(-jnp.abs(gs))
                z = t_ / (2.0 + t_)
                z2 = z * z
                ln = (2.0 * z) * (1.0 + z2 * (
                    1.0 / 3.0 + z2 * (1.0 / 5.0 + z2 * (
                        1.0 / 7.0 + z2 * (1.0 / 9.0 + z2 * (1.0 / 11.0))))))
                sp = jnp.maximum(gs, 0.0) + ln
                mBuf[i] = sig * sp

            hM[k] = pltpu.async_copy(mBuf, agg.at[dIdx.at[k]], semM[b],
                                     add=True)
        for k in range(max(0, SCH - DEPTH), SCH):
            hM[k].wait()
        return 0
    lax.fori_loop(0, NSCH, _schunk, 0)

    plsc.subcore_barrier()
    _read_acc(cid, sid, zBuf, agg, out)


@functools.partial(
    pl.kernel,
    mesh=_SC_MESH,
    out_type=jax.ShapeDtypeStruct((2, N, FEA), jnp.float32),
    compiler_params=pltpu.CompilerParams(use_tc_tiling_on_sc=False),
    scratch_types=[
        pltpu.VMEM((SCH, CHUNK), jnp.int32),
        pltpu.VMEM((CHUNK, FEA), jnp.float32),
        pltpu.VMEM((ROWS_OUT, FEA), jnp.float32),
        pltpu.VMEM_SHARED((N, FEA), jnp.float32),
    ],
)
def _sc_cnt(dst3d, out, dIdx, oBuf, zBuf, agg):
    cid = lax.axis_index("c")
    sid = lax.axis_index("s")
    wid = sid * 2 + cid

    _zero_acc(sid, zBuf, agg)

    def _orow(k, _):
        oBuf[k] = jnp.ones((FEA,), jnp.float32)
        return 0
    lax.fori_loop(0, CHUNK, _orow, 0)
    plsc.subcore_barrier()

    t0 = wid * NSCH

    def _schunk(t, _):
        pltpu.sync_copy(dst3d.at[t0 + t], dIdx)

        def _chunk(k, _):
            pltpu.sync_copy(oBuf, agg.at[dIdx.at[k]], add=True)
            return 0
        lax.fori_loop(0, SCH, _chunk, 0)
        return 0
    lax.fori_loop(0, NSCH, _schunk, 0)

    plsc.subcore_barrier()
    _read_acc(cid, sid, zBuf, agg, out)


def _row(b):
    return jnp.reshape(b, (1, -1))


def kernel(x, edge_index, edge_attr, batch, params):
    p = params
    convs = p['convs']
    node_in = x.shape[1]
    edge_in = edge_attr.shape[1]

    src3d = edge_index[0].reshape(E // (SCH * CHUNK), SCH, CHUNK)
    dst3d = edge_index[1].reshape(E // (SCH * CHUNK), SCH, CHUNK)

    wd_l = [jnp.concatenate([c['wf'][:, :FEA].T, c['ws'][:, :FEA].T], axis=1)
            for c in convs]
    ws_l = [jnp.concatenate([c['wf'][:, FEA:2 * FEA].T,
                             c['ws'][:, FEA:2 * FEA].T], axis=1)
            for c in convs]

    BLK_N = 2000
    h0, pdqd, psqs = pl.pallas_call(
        _node_prep_body,
        grid=(N // BLK_N,),
        in_specs=[pl.BlockSpec((BLK_N, node_in), lambda i: (i, 0)),
                  pl.BlockSpec((node_in, FEA), lambda i: (0, 0)),
                  pl.BlockSpec((1, FEA), lambda i: (0, 0)),
                  pl.BlockSpec((FEA, 2 * FEA), lambda i: (0, 0)),
                  pl.BlockSpec((FEA, 2 * FEA), lambda i: (0, 0))],
        out_specs=[pl.BlockSpec((BLK_N, FEA), lambda i: (i, 0)),
                   pl.BlockSpec((BLK_N, 2 * FEA), lambda i: (i, 0)),
                   pl.BlockSpec((BLK_N, 2 * FEA), lambda i: (i, 0))],
        out_shape=[jax.ShapeDtypeStruct((N, FEA), jnp.float32),
                   jax.ShapeDtypeStruct((N, 2 * FEA), jnp.float32),
                   jax.ShapeDtypeStruct((N, 2 * FEA), jnp.float32)],
    )(x, p['node_w'].T, _row(p['node_b']), wd_l[0], ws_l[0])

    BLK_E = 4000
    fs = []
    for c in convs:
        f_l, s_l = pl.pallas_call(
            _edge_prep_body,
            grid=(E // BLK_E,),
            in_specs=[pl.BlockSpec((BLK_E, edge_in), lambda i: (i, 0)),
                      pl.BlockSpec((edge_in, FEA), lambda i: (0, 0)),
                      pl.BlockSpec((1, FEA), lambda i: (0, 0)),
                      pl.BlockSpec((FEA, FEA), lambda i: (0, 0)),
                      pl.BlockSpec((1, FEA), lambda i: (0, 0)),
                      pl.BlockSpec((FEA, FEA), lambda i: (0, 0)),
                      pl.BlockSpec((1, FEA), lambda i: (0, 0))],
            out_specs=[pl.BlockSpec((BLK_E, FEA), lambda i: (i, 0)),
                       pl.BlockSpec((BLK_E, FEA), lambda i: (i, 0))],
            out_shape=[jax.ShapeDtypeStruct((E, FEA), jnp.float32),
                       jax.ShapeDtypeStruct((E, FEA), jnp.float32)],
        )(edge_attr, p['edge_w'].T, _row(p['edge_b']),
          c['wf'][:, 2 * FEA:].T, _row(c['bf']),
          c['ws'][:, 2 * FEA:].T, _row(c['bs']))
        fs.append((f_l, s_l))

    cntp = _sc_cnt(dst3d)

    h = h0
    for l in range(N_LAYERS):
        c = convs[l]
        aggp = _sc_layer(dst3d, src3d, pdqd, psqs, fs[l][0], fs[l][1])
        if l < N_LAYERS - 1:
            h, pdqd, psqs = pl.pallas_call(
                _post_body,
                out_shape=[jax.ShapeDtypeStruct((N, FEA), jnp.float32),
                           jax.ShapeDtypeStruct((N, 2 * FEA), jnp.float32),
                           jax.ShapeDtypeStruct((N, 2 * FEA), jnp.float32)],
            )(aggp, cntp, h, _row(c['gamma']), _row(c['beta']),
              wd_l[l + 1], ws_l[l + 1])
        else:
            y = pl.pallas_call(
                _post_pool_body,
                out_shape=jax.ShapeDtypeStruct((G, 1), jnp.float32),
            )(aggp, cntp, h, _row(c['gamma']), _row(c['beta']),
              batch.reshape(N, 1),
              p['gate_w1'].T, _row(p['gate_b1']),
              p['gate_w2'].T, _row(p['gate_b2']),
              p['nn_w1'].T, _row(p['nn_b1']),
              p['nn_w2'].T, _row(p['nn_b2']),
              p['h_w'].T, _row(p['h_b']),
              p['out_w'].T, _row(p['out_b']))
    return jnp.reshape(y, (G,))
